# baseline (speedup 1.0000x reference)
import functools

import jax
import jax.numpy as jnp
from jax.experimental import pallas as pl
from jax.experimental.pallas import tpu as pltpu
from jax.experimental.pallas import tpu_sc as plsc

_FP = 128


_W = 80


def _sc_gather_pair(table_lo, table_hi, idx_lo, idx_hi):
    nblk = idx_lo.shape[0]
    m2 = nblk * _W
    mesh = plsc.VectorSubcoreMesh(core_axis_name="core", subcore_axis_name="subcore")
    row_ty = jax.ShapeDtypeStruct((m2, _FP), table_lo.dtype)

    @functools.partial(pl.kernel, out_type=(row_ty, row_ty), mesh=mesh)
    def gather_kernel(tlo_hbm, thi_hbm, ilo_hbm, ihi_hbm, olo_hbm, ohi_hbm):
        def body(ilo_v, ihi_v, olo_v, ohi_v):
            pltpu.sync_copy(tlo_hbm.at[ilo_v.at[0, 0]], olo_v)
            pltpu.sync_copy(thi_hbm.at[ihi_v.at[0, 0]], ohi_v)

        pltpu.emit_pipeline(
            body,
            grid=(nblk,),
            in_specs=[
                pl.BlockSpec((1, 1, _W), index_map=lambda i: (i, 0, 0)),
                pl.BlockSpec((1, 1, _W), index_map=lambda i: (i, 0, 0)),
            ],
            out_specs=[
                pl.BlockSpec((_W, _FP), index_map=lambda i: (i, 0)),
                pl.BlockSpec((_W, _FP), index_map=lambda i: (i, 0)),
            ],
            core_axis_name=("core", "subcore"),
            dimension_semantics=(pltpu.PARALLEL,),
        )(ilo_hbm, ihi_hbm, olo_hbm, ohi_hbm)

    return gather_kernel(table_lo, table_hi, idx_lo, idx_hi)


def _add_body(x_ref, lo_ref, hi_ref, o_ref):
    o_ref[...] = x_ref[...] + (lo_ref[...] + hi_ref[...])


def _tc_broadcast_add(xv, lo4, hi4):
    b, t, n2, fp = xv.shape
    rows = 4
    assert t % rows == 0
    grid = (b, t // rows)
    return pl.pallas_call(
        _add_body,
        grid=grid,
        in_specs=[
            pl.BlockSpec((1, rows, n2, fp), lambda i, j: (i, j, 0, 0)),
            pl.BlockSpec((1, 1, n2, fp), lambda i, j: (i, 0, 0, 0)),
            pl.BlockSpec((1, 1, n2, fp), lambda i, j: (i, 0, 0, 0)),
        ],
        out_specs=pl.BlockSpec((1, rows, n2, fp), lambda i, j: (i, j, 0, 0)),
        out_shape=jax.ShapeDtypeStruct(xv.shape, xv.dtype),
        compiler_params=pltpu.CompilerParams(
            dimension_semantics=("parallel", "arbitrary"),
        ),
    )(xv, lo4, hi4)


def kernel(degree, x, emb_table):
    b, t, n, f = x.shape
    assert n % 2 == 0 and 2 * f == _FP
    n2 = n // 2
    m2 = b * n2
    assert m2 % _W == 0, m2
    table_lo = jnp.pad(emb_table, ((0, 0), (0, f)))
    table_hi = jnp.pad(emb_table, ((0, 0), (f, 0)))
    deg = degree.astype(jnp.int32)
    d_lo = deg[:, 0::2].reshape(m2 // _W, 1, _W)
    d_hi = deg[:, 1::2].reshape(m2 // _W, 1, _W)
    demb_lo, demb_hi = _sc_gather_pair(table_lo, table_hi, d_lo, d_hi)
    lo4 = demb_lo.reshape(b, 1, n2, _FP)
    hi4 = demb_hi.reshape(b, 1, n2, _FP)
    xv = x.reshape(b, t, n2, _FP)
    out = _tc_broadcast_add(xv, lo4, hi4)
    return out.reshape(x.shape)
